# baseline (speedup 1.0000x reference)
import functools

import jax
import jax.numpy as jnp
from jax import lax
from jax.experimental import pallas as pl
from jax.experimental.pallas import tpu as pltpu
from jax.experimental.pallas import tpu_sc as plsc

_NC = 2
_NS = 16
_NW = _NC * _NS
_CK = 128
_EB = _NW * _CK


def _pad16(v):
    return ((v + 127) // 128) * 128


@functools.lru_cache(maxsize=None)
def _sc_scatter_fn(n_in_pad, n_out_pad, feat, n_chunks):
    rz = n_out_pad // _NS
    nb = max(1, min(8, (256 * 1024) // (_CK * feat * 4)))
    while n_chunks % nb:
        nb //= 2
    n_groups = n_chunks // nb
    mesh = plsc.VectorSubcoreMesh(core_axis_name="c", subcore_axis_name="s")

    @functools.partial(
        pl.kernel,
        mesh=mesh,
        compiler_params=pltpu.CompilerParams(use_tc_tiling_on_sc=False),
        out_type=jax.ShapeDtypeStruct((_NC, n_out_pad, feat), jnp.float32),
        scratch_types=[
            pltpu.VMEM((n_chunks, _CK), jnp.int32),
            pltpu.VMEM((n_chunks, _CK), jnp.int32),
            pltpu.VMEM((nb * _CK, feat), jnp.float32),
            pltpu.VMEM_SHARED((n_out_pad, feat), jnp.float32),
            pltpu.SemaphoreType.DMA,
            pltpu.SemaphoreType.DMA,
        ],
    )
    def k(table, src, dst, zeros, out, idx_s, idx_d, rows, accum, gsem, ssem):
        cid = lax.axis_index("c")
        sid = lax.axis_index("s")
        wid = sid * _NC + cid
        pltpu.sync_copy(src.at[wid], idx_s)
        pltpu.sync_copy(dst.at[wid], idx_d)
        r0 = sid * rz
        pltpu.sync_copy(zeros.at[pl.ds(r0, rz)], accum.at[pl.ds(r0, rz)])
        plsc.subcore_barrier()

        @pl.loop(0, n_groups)
        def _(g):
            base = g * nb
            ghs = [
                pltpu.async_copy(table.at[idx_s.at[base + b]],
                                 rows.at[pl.ds(b * _CK, _CK)], gsem)
                for b in range(nb)
            ]
            for h in ghs:
                h.wait()
            shs = [
                pltpu.async_copy(rows.at[pl.ds(b * _CK, _CK)],
                                 accum.at[idx_d.at[base + b]], ssem, add=True)
                for b in range(nb)
            ]
            for h in shs:
                h.wait()

        plsc.subcore_barrier()
        pltpu.sync_copy(accum.at[pl.ds(r0, rz)], out.at[cid].at[pl.ds(r0, rz)])

    return k


def _sc_scatter(table_pad, src3, dst3, n_out_pad):
    feat = table_pad.shape[1]
    zeros = jnp.zeros((n_out_pad, feat), jnp.float32)
    fn = _sc_scatter_fn(table_pad.shape[0], n_out_pad, feat, src3.shape[1])
    return fn(table_pad, src3, dst3, zeros)


def _prep_idx(idx, fill):
    e = idx.shape[0]
    gran = _EB * 8
    e_pad = ((e + gran - 1) // gran) * gran
    out = jnp.full((e_pad,), fill, jnp.int32).at[:e].set(idx.astype(jnp.int32))
    return out.reshape(_NW, e_pad // _EB, _CK)


def _elu(z):
    return jnp.where(z > 0, z, jnp.exp(jnp.minimum(z, 0.0)) - 1.0)


def _tc(f, shape, *args):
    return pl.pallas_call(
        f, out_shape=jax.ShapeDtypeStruct(shape, jnp.float32))(*args)


def _mm_scale(x, w, dinv):
    def f(x_ref, w_ref, d_ref, o_ref):
        o_ref[...] = jnp.dot(
            x_ref[...], w_ref[...], preferred_element_type=jnp.float32
        ) * d_ref[...]
    return _tc(f, (x.shape[0], w.shape[1]), x, w, dinv)


def _gcn_combine(parts, hprime, dinv, b):
    n = hprime.shape[0]

    def f(p_ref, h_ref, d_ref, b_ref, o_ref):
        s = (p_ref[0] + p_ref[1])[:n, :]
        z = d_ref[...] * (s + h_ref[...]) + b_ref[...]
        o_ref[...] = _elu(z)
    return _tc(f, hprime.shape, parts, hprime, dinv, b.reshape(1, -1))


def _deg_combine(parts):
    def f(p_ref, o_ref):
        deg = p_ref[0] + p_ref[1] + 1.0
        o_ref[...] = lax.rsqrt(deg)
    return _tc(f, parts.shape[1:], parts)


def _matvec(x, p):
    p8 = jnp.zeros((p.shape[0], 8), jnp.float32).at[:, 0].set(p)
    nrm = jnp.sqrt(jnp.sum(p * p)).reshape(1, 1)

    def f(x_ref, p_ref, n_ref, o_ref):
        o_ref[...] = jnp.dot(
            x_ref[...], p_ref[...], preferred_element_type=jnp.float32
        ) / n_ref[...]
    return _tc(f, (x.shape[0], 8), x, p8, nrm)[:, 0]


def _pool_combine(parts, top_vals):
    k = top_vals.shape[0]

    def f(p_ref, t_ref, o_ref):
        s = (p_ref[0] + p_ref[1])[:k, :]
        o_ref[...] = _elu(s * jax.nn.sigmoid(t_ref[...]))
    return _tc(f, (k, parts.shape[2]), parts, top_vals.reshape(k, 1))


def _unpool_cat(parts, x_skip):
    n = x_skip.shape[0]
    f1 = parts.shape[2]

    def f(p_ref, x_ref, o_ref):
        u = (p_ref[0] + p_ref[1])[:n, :]
        o_ref[...] = _elu(jnp.concatenate([u, x_ref[...]], axis=1))
    return _tc(f, (n, f1 + x_skip.shape[1]), parts, x_skip)


def _readout(x, batch, num_graphs):
    n, feat = x.shape

    def f(x_ref, b_ref, o_ref):
        xv = x_ref[...]
        bv = b_ref[...]
        onehot = (bv == lax.broadcasted_iota(jnp.int32, (1, num_graphs), 1)
                  ).astype(jnp.float32)
        ssum = jnp.dot(onehot.T, xv, preferred_element_type=jnp.float32)
        cnt = jnp.sum(onehot, axis=0).reshape(num_graphs, 1)
        mean = ssum / jnp.maximum(cnt, 1.0)

        def body(g, _):
            mask = bv == g
            m = jnp.max(jnp.where(mask, xv, -jnp.inf), axis=0)
            o_ref[pl.ds(g, 1), :feat] = m.reshape(1, feat)
            return 0
        lax.fori_loop(0, num_graphs, body, 0)
        o_ref[:, feat:] = mean
    return _tc(f, (num_graphs, 2 * feat), x, batch.reshape(n, 1))


def _mlp_head(gcat, l1w, clsw, clsb):
    def f(g_ref, w1_ref, w2_ref, b2_ref, o_ref):
        g = _elu(g_ref[...])
        g = _elu(jnp.dot(g, w1_ref[...], preferred_element_type=jnp.float32))
        lg = jnp.dot(g, w2_ref[...], preferred_element_type=jnp.float32) \
            + b2_ref[...]
        m = jnp.max(lg, axis=1, keepdims=True)
        sh = lg - m
        o_ref[...] = sh - jnp.log(jnp.sum(jnp.exp(sh), axis=1, keepdims=True))
    return _tc(f, (gcat.shape[0], clsw.shape[1]), gcat, l1w, clsw,
               clsb.reshape(1, -1))


def _degree_inv(dst3, n):
    n_pad = _pad16(n + 1)
    ones_tab = jnp.ones((n_pad, 8), jnp.float32)
    parts = _sc_scatter(ones_tab, dst3, dst3, n_pad)
    return _deg_combine(parts)[:n, :1]


def _gcn(x, w, b, src3, dst3, dinv, n):
    n_pad = _pad16(n + 1)
    hprime = _mm_scale(x, w, dinv)
    tab = jnp.zeros((n_pad, w.shape[1]), jnp.float32).at[:n].set(hprime)
    parts = _sc_scatter(tab, src3, dst3, n_pad)
    return _gcn_combine(parts, hprime, dinv, b)


def _pool(x, src, dst, p, n):
    k = n // 2
    feat = x.shape[1]
    y = _matvec(x, p)
    top_vals, top_idx = lax.top_k(y, k)
    n_pad = _pad16(n + 1)
    k_pad = _pad16(k + 1)
    tab = jnp.zeros((n_pad, feat), jnp.float32).at[:n].set(x)
    src3 = _prep_idx(top_idx, n)
    dst3 = _prep_idx(jnp.arange(k, dtype=jnp.int32), k)
    parts = _sc_scatter(tab, src3, dst3, k_pad)
    xp = _pool_combine(parts, top_vals)
    new_idx = jnp.full((n + 1,), -1, jnp.int32).at[top_idx].set(
        jnp.arange(k, dtype=jnp.int32))
    s2 = new_idx[src]
    d2 = new_idx[dst]
    mask = (s2 >= 0) & (d2 >= 0)
    sentinel = k * k
    lin = jnp.where(mask, s2 * k + d2, sentinel)
    lin_sorted = jnp.sort(lin)
    is_first = jnp.concatenate(
        [jnp.ones((1,), jnp.bool_), lin_sorted[1:] != lin_sorted[:-1]])
    keep = is_first & (lin_sorted < sentinel)
    s_out = jnp.where(keep, lin_sorted // k, k).astype(jnp.int32)
    d_out = jnp.where(keep, lin_sorted % k, k).astype(jnp.int32)
    return xp, s_out, d_out, top_idx


def _unpool(x_small, idx, x_skip, n):
    k, feat = x_small.shape
    n_pad = _pad16(n + 1)
    k_pad = _pad16(k + 1)
    tab = jnp.zeros((k_pad, feat), jnp.float32).at[:k].set(x_small)
    src3 = _prep_idx(jnp.arange(k, dtype=jnp.int32), k)
    dst3 = _prep_idx(idx.astype(jnp.int32), n)
    parts = _sc_scatter(tab, src3, dst3, n_pad)
    return _unpool_cat(parts, x_skip)


def kernel(x, edge_index, batch, W1, b1, W2, b2, W3, b3, W4, b4, W5, b5,
           W6, b6, W7, b7, p1, p2, p3, l1W, clsW, clsb):
    n1 = x.shape[0]
    n2, n3, n4 = n1 // 2, n1 // 4, n1 // 8
    num_graphs = 64

    src1, dst1 = edge_index[0], edge_index[1]
    s1_3, d1_3 = _prep_idx(src1, n1), _prep_idx(dst1, n1)
    dinv1 = _degree_inv(d1_3, n1)

    x1 = _gcn(x, W1, b1, s1_3, d1_3, dinv1, n1)
    x2, src2, dst2, idx2 = _pool(x1, src1, dst1, p1, n1)
    s2_3, d2_3 = _prep_idx(src2, n2), _prep_idx(dst2, n2)
    dinv2 = _degree_inv(d2_3, n2)

    x3 = _gcn(x2, W2, b2, s2_3, d2_3, dinv2, n2)
    x4, src4, dst4, idx4 = _pool(x3, src2, dst2, p2, n2)
    s4_3, d4_3 = _prep_idx(src4, n3), _prep_idx(dst4, n3)
    dinv3 = _degree_inv(d4_3, n3)

    x5 = _gcn(x4, W3, b3, s4_3, d4_3, dinv3, n3)
    x6, src6, dst6, idx6 = _pool(x5, src4, dst4, p3, n3)
    s6_3, d6_3 = _prep_idx(src6, n4), _prep_idx(dst6, n4)
    dinv4 = _degree_inv(d6_3, n4)

    x7 = _gcn(x6, W4, b4, s6_3, d6_3, dinv4, n4)
    x8 = _unpool(x7, idx6, x5, n3)
    x9 = _gcn(x8, W5, b5, s4_3, d4_3, dinv3, n3)
    x10 = _unpool(x9, idx4, x3, n2)
    x11 = _gcn(x10, W6, b6, s2_3, d2_3, dinv2, n2)
    x12 = _unpool(x11, idx2, x1, n1)
    x13 = _gcn(x12, W7, b7, s1_3, d1_3, dinv1, n1)

    gcat = _readout(x13, batch, num_graphs)
    return _mlp_head(gcat, l1W, clsW, clsb)


# speedup vs baseline: 4.7113x; 4.7113x over previous
import functools

import jax
import jax.numpy as jnp
from jax import lax
from jax.experimental import pallas as pl
from jax.experimental.pallas import tpu as pltpu
from jax.experimental.pallas import tpu_sc as plsc

_NC = 2
_NS = 16
_NW = _NC * _NS
_CK = 128
_EB = _NW * _CK


def _pad16(v):
    return ((v + 127 + 128) // 128) * 128


@functools.lru_cache(maxsize=None)
def _sc_scatter_fn(n_in_pad, n_out_pad, feat, n_chunks):
    rz = n_out_pad // _NS
    nb = max(1, min(8, (256 * 1024) // (_CK * feat * 4)))
    while n_chunks % nb:
        nb //= 2
    n_groups = n_chunks // nb
    n_fv = -(-n_groups // 16)
    mesh = plsc.VectorSubcoreMesh(core_axis_name="c", subcore_axis_name="s")

    @functools.partial(
        pl.kernel,
        mesh=mesh,
        compiler_params=pltpu.CompilerParams(use_tc_tiling_on_sc=False, needs_layout_passes=False),
        out_type=jax.ShapeDtypeStruct((_NC, n_out_pad, feat), jnp.float32),
        scratch_types=[
            pltpu.VMEM((n_chunks, _CK), jnp.int32),
            pltpu.VMEM((n_chunks, _CK), jnp.int32),
            pltpu.VMEM((nb * _CK, feat), jnp.float32),
            pltpu.VMEM((n_fv, 16), jnp.int32),
            pltpu.VMEM_SHARED((n_out_pad, feat), jnp.float32),
            pltpu.SemaphoreType.DMA,
            pltpu.SemaphoreType.DMA,
        ],
    )
    def k(table, src, dst, flags, zeros, out,
          idx_s, idx_d, rows, fvm, accum, gsem, ssem):
        cid = lax.axis_index("c")
        sid = lax.axis_index("s")
        wid = sid * _NC + cid
        pltpu.sync_copy(src.at[wid], idx_s)
        pltpu.sync_copy(dst.at[wid], idx_d)
        pltpu.sync_copy(flags.at[wid], fvm)
        r0 = sid * rz
        pltpu.sync_copy(zeros.at[pl.ds(r0, rz)], accum.at[pl.ds(r0, rz)])
        cnt = plsc.all_reduce_population_count(fvm[0] > 0)
        for j in range(1, n_fv):
            cnt = cnt + plsc.all_reduce_population_count(fvm[j] > 0)
        n_my_groups = jnp.max(cnt)
        plsc.subcore_barrier()

        @pl.loop(0, n_my_groups)
        def _(g):
            base = g * nb
            ghs = [
                pltpu.async_copy(table.at[idx_s.at[base + b]],
                                 rows.at[pl.ds(b * _CK, _CK)], gsem)
                for b in range(nb)
            ]
            for h in ghs:
                h.wait()
            shs = [
                pltpu.async_copy(rows.at[pl.ds(b * _CK, _CK)],
                                 accum.at[idx_d.at[base + b]], ssem, add=True)
                for b in range(nb)
            ]
            for h in shs:
                h.wait()

        plsc.subcore_barrier()
        pltpu.sync_copy(accum.at[pl.ds(r0, rz)], out.at[cid].at[pl.ds(r0, rz)])

    return k


def _sc_scatter(table_pad, src3, dst3, n_out_pad, n_valid):
    feat = table_pad.shape[1]
    n_chunks = src3.shape[1]
    nb = max(1, min(8, (256 * 1024) // (_CK * feat * 4)))
    while n_chunks % nb:
        nb //= 2
    flags = _group_flags(n_valid, n_chunks, nb)
    zeros = jnp.zeros((n_out_pad, feat), jnp.float32)
    fn = _sc_scatter_fn(table_pad.shape[0], n_out_pad, feat, n_chunks)
    return fn(table_pad, src3, dst3, flags, zeros)


def _prep_idx(idx, fill):
    e = idx.shape[0]
    gran = _EB * 8
    e_pad = ((e + gran - 1) // gran) * gran
    spread = fill + (jnp.arange(e_pad, dtype=jnp.int32) % 128)
    out = spread.at[:e].set(idx.astype(jnp.int32))
    c = e_pad // _EB
    return out.reshape(c, _NW, _CK).transpose(1, 0, 2)


def _group_flags(n_valid, n_chunks, nb):
    g_max = n_chunks // nb
    n_fv = -(-g_max // 16)
    nvc = (n_valid + _CK - 1) // _CK
    w = jnp.arange(_NW, dtype=jnp.int32)
    trips = jnp.maximum(0, (nvc - w + _NW - 1) // _NW)
    groups = (trips + nb - 1) // nb
    j = jnp.arange(n_fv * 16, dtype=jnp.int32)
    return (j[None, :] < groups[:, None]).astype(jnp.int32).reshape(
        _NW, n_fv, 16)


def _elu(z):
    return jnp.where(z > 0, z, jnp.exp(jnp.minimum(z, 0.0)) - 1.0)


def _tc(f, shape, *args):
    return pl.pallas_call(
        f, out_shape=jax.ShapeDtypeStruct(shape, jnp.float32))(*args)


def _mm_scale(x, w, dinv):
    def f(x_ref, w_ref, d_ref, o_ref):
        o_ref[...] = jnp.dot(
            x_ref[...], w_ref[...], preferred_element_type=jnp.float32
        ) * d_ref[...]
    return _tc(f, (x.shape[0], w.shape[1]), x, w, dinv)


def _gcn_combine(parts, hprime, dinv, b):
    n = hprime.shape[0]

    def f(p_ref, h_ref, d_ref, b_ref, o_ref):
        s = (p_ref[0] + p_ref[1])[:n, :]
        z = d_ref[...] * (s + h_ref[...]) + b_ref[...]
        o_ref[...] = _elu(z)
    return _tc(f, hprime.shape, parts, hprime, dinv, b.reshape(1, -1))


def _deg_combine(parts):
    def f(p_ref, o_ref):
        deg = p_ref[0] + p_ref[1] + 1.0
        o_ref[...] = lax.rsqrt(deg)
    return _tc(f, parts.shape[1:], parts)


def _matvec(x, p):
    p8 = jnp.zeros((p.shape[0], 8), jnp.float32).at[:, 0].set(p)
    nrm = jnp.sqrt(jnp.sum(p * p)).reshape(1, 1)

    def f(x_ref, p_ref, n_ref, o_ref):
        o_ref[...] = jnp.dot(
            x_ref[...], p_ref[...], preferred_element_type=jnp.float32
        ) / n_ref[...]
    return _tc(f, (x.shape[0], 8), x, p8, nrm)[:, 0]


def _pool_combine(parts, top_vals):
    k = top_vals.shape[0]

    def f(p_ref, t_ref, o_ref):
        s = (p_ref[0] + p_ref[1])[:k, :]
        o_ref[...] = _elu(s * jax.nn.sigmoid(t_ref[...]))
    return _tc(f, (k, parts.shape[2]), parts, top_vals.reshape(k, 1))


def _unpool_cat(parts, x_skip):
    n = x_skip.shape[0]
    f1 = parts.shape[2]

    def f(p_ref, x_ref, o_ref):
        u = (p_ref[0] + p_ref[1])[:n, :]
        o_ref[...] = _elu(jnp.concatenate([u, x_ref[...]], axis=1))
    return _tc(f, (n, f1 + x_skip.shape[1]), parts, x_skip)


def _readout(x, batch, num_graphs):
    n, feat = x.shape

    def f(x_ref, b_ref, o_ref):
        xv = x_ref[...]
        bv = b_ref[...]
        onehot = (bv == lax.broadcasted_iota(jnp.int32, (1, num_graphs), 1)
                  ).astype(jnp.float32)
        ssum = jnp.dot(onehot.T, xv, preferred_element_type=jnp.float32)
        cnt = jnp.sum(onehot, axis=0).reshape(num_graphs, 1)
        mean = ssum / jnp.maximum(cnt, 1.0)

        def body(g, _):
            mask = bv == g
            m = jnp.max(jnp.where(mask, xv, -jnp.inf), axis=0)
            o_ref[pl.ds(g, 1), :feat] = m.reshape(1, feat)
            return 0
        lax.fori_loop(0, num_graphs, body, 0)
        o_ref[:, feat:] = mean
    return _tc(f, (num_graphs, 2 * feat), x, batch.reshape(n, 1))


def _mlp_head(gcat, l1w, clsw, clsb):
    def f(g_ref, w1_ref, w2_ref, b2_ref, o_ref):
        g = _elu(g_ref[...])
        g = _elu(jnp.dot(g, w1_ref[...], preferred_element_type=jnp.float32))
        lg = jnp.dot(g, w2_ref[...], preferred_element_type=jnp.float32) \
            + b2_ref[...]
        m = jnp.max(lg, axis=1, keepdims=True)
        sh = lg - m
        o_ref[...] = sh - jnp.log(jnp.sum(jnp.exp(sh), axis=1, keepdims=True))
    return _tc(f, (gcat.shape[0], clsw.shape[1]), gcat, l1w, clsw,
               clsb.reshape(1, -1))


def _degree_inv(dst3, n, n_valid):
    n_pad = _pad16(n + 1)
    ones_tab = jnp.ones((n_pad, 8), jnp.float32)
    parts = _sc_scatter(ones_tab, dst3, dst3, n_pad, n_valid)
    return _deg_combine(parts)[:n, :1]


def _gcn(x, w, b, src3, dst3, dinv, n, n_valid):
    n_pad = _pad16(n + 1)
    hprime = _mm_scale(x, w, dinv)
    tab = jnp.zeros((n_pad, w.shape[1]), jnp.float32).at[:n].set(hprime)
    parts = _sc_scatter(tab, src3, dst3, n_pad, n_valid)
    return _gcn_combine(parts, hprime, dinv, b)


def _pool(x, src, dst, p, n):
    k = n // 2
    feat = x.shape[1]
    y = _matvec(x, p)
    top_vals, top_idx = lax.top_k(y, k)
    n_pad = _pad16(n + 1)
    k_pad = _pad16(k + 1)
    tab = jnp.zeros((n_pad, feat), jnp.float32).at[:n].set(x)
    src3 = _prep_idx(top_idx, n)
    dst3 = _prep_idx(jnp.arange(k, dtype=jnp.int32), k)
    parts = _sc_scatter(tab, src3, dst3, k_pad, k)
    xp = _pool_combine(parts, top_vals)
    new_idx = jnp.full((n + 1,), -1, jnp.int32).at[top_idx].set(
        jnp.arange(k, dtype=jnp.int32))
    s2 = new_idx[src]
    d2 = new_idx[dst]
    mask = (s2 >= 0) & (d2 >= 0)
    sentinel = k * k
    lin = jnp.where(mask, s2 * k + d2, sentinel)
    lin_sorted = jnp.sort(lin)
    is_first = jnp.concatenate(
        [jnp.ones((1,), jnp.bool_), lin_sorted[1:] != lin_sorted[:-1]])
    keep = is_first & (lin_sorted < sentinel)
    n_valid = jnp.sum(keep.astype(jnp.int32))
    junk = k + (jnp.arange(lin.shape[0], dtype=jnp.int32) % 128)
    s_out = jnp.where(keep, lin_sorted // k, junk).astype(jnp.int32)
    d_out = jnp.where(keep, lin_sorted % k, junk).astype(jnp.int32)
    return xp, s_out, d_out, top_idx, n_valid


def _unpool(x_small, idx, x_skip, n):
    k, feat = x_small.shape
    n_pad = _pad16(n + 1)
    k_pad = _pad16(k + 1)
    tab = jnp.zeros((k_pad, feat), jnp.float32).at[:k].set(x_small)
    src3 = _prep_idx(jnp.arange(k, dtype=jnp.int32), k)
    dst3 = _prep_idx(idx.astype(jnp.int32), n)
    parts = _sc_scatter(tab, src3, dst3, n_pad, k)
    return _unpool_cat(parts, x_skip)


def kernel(x, edge_index, batch, W1, b1, W2, b2, W3, b3, W4, b4, W5, b5,
           W6, b6, W7, b7, p1, p2, p3, l1W, clsW, clsb):
    n1 = x.shape[0]
    n2, n3, n4 = n1 // 2, n1 // 4, n1 // 8
    num_graphs = 64

    src1, dst1 = edge_index[0], edge_index[1]
    ne = src1.shape[0]
    s1_3, d1_3 = _prep_idx(src1, n1), _prep_idx(dst1, n1)
    dinv1 = _degree_inv(d1_3, n1, ne)

    x1 = _gcn(x, W1, b1, s1_3, d1_3, dinv1, n1, ne)
    x2, src2, dst2, idx2, nv2 = _pool(x1, src1, dst1, p1, n1)
    s2_3, d2_3 = _prep_idx(src2, n2), _prep_idx(dst2, n2)
    dinv2 = _degree_inv(d2_3, n2, nv2)

    x3 = _gcn(x2, W2, b2, s2_3, d2_3, dinv2, n2, nv2)
    x4, src4, dst4, idx4, nv4 = _pool(x3, src2, dst2, p2, n2)
    s4_3, d4_3 = _prep_idx(src4, n3), _prep_idx(dst4, n3)
    dinv3 = _degree_inv(d4_3, n3, nv4)

    x5 = _gcn(x4, W3, b3, s4_3, d4_3, dinv3, n3, nv4)
    x6, src6, dst6, idx6, nv6 = _pool(x5, src4, dst4, p3, n3)
    s6_3, d6_3 = _prep_idx(src6, n4), _prep_idx(dst6, n4)
    dinv4 = _degree_inv(d6_3, n4, nv6)

    x7 = _gcn(x6, W4, b4, s6_3, d6_3, dinv4, n4, nv6)
    x8 = _unpool(x7, idx6, x5, n3)
    x9 = _gcn(x8, W5, b5, s4_3, d4_3, dinv3, n3, nv4)
    x10 = _unpool(x9, idx4, x3, n2)
    x11 = _gcn(x10, W6, b6, s2_3, d2_3, dinv2, n2, nv2)
    x12 = _unpool(x11, idx2, x1, n1)
    x13 = _gcn(x12, W7, b7, s1_3, d1_3, dinv1, n1, ne)

    gcat = _readout(x13, batch, num_graphs)
    return _mlp_head(gcat, l1W, clsW, clsb)
